# baseline (speedup 1.0000x reference)
import functools

import numpy as np
import jax
import jax.numpy as jnp
from jax import lax
from jax.experimental import pallas as pl
from jax.experimental.pallas import tpu as pltpu
from jax.experimental.pallas import tpu_sc as plsc


_RANDI = np.array([0, 3, 5, 7, 8, 11, 12, 15, 16, 19, 20, 23, 24, 27, 29, 30,
                   33, 35, 37, 39, 41, 43, 44, 47, 49, 51, 53, 54, 56, 59, 60,
                   63], dtype=np.int32)
_RANDJ = np.array([1, 2, 4, 6, 9, 10, 13, 14, 16, 19, 20, 22, 24, 27, 29, 30,
                   33, 34, 37, 39, 41, 42, 44, 46, 48, 50, 52, 54, 56, 59, 60,
                   62], dtype=np.int32)

_B, _C, _H, _W = 1024, 16, 64, 64
_BC = _B * _C
_NROWS = _BC * 32
_NW = 32
_RPW = _NROWS // _NW
_K = 128
_NCH = _RPW // _K

_MASK_LO = int(sum((int(_RANDI[i]) - 2 * i) << i for i in range(16)))
_MASK_HI = int(sum((int(_RANDI[16 + i]) - 2 * (16 + i)) << i for i in range(16)))

_mesh = plsc.VectorSubcoreMesh(core_axis_name="c", subcore_axis_name="s")


@functools.partial(
    pl.kernel,
    out_type=jax.ShapeDtypeStruct((_NROWS, 32), jnp.float32),
    mesh=_mesh,
    compiler_params=pltpu.CompilerParams(needs_layout_passes=False,
                                         use_tc_tiling_on_sc=False),
    scratch_types=[
        pltpu.VMEM((_K,), jnp.int32),
        pltpu.VMEM((_K, _W), jnp.float32),
        pltpu.VMEM((_K, 32), jnp.float32),
        pltpu.SemaphoreType.DMA,
    ],
)
def _sc_gather(table, idxs, out, idx_v, rows_v, out_v, sem):
    wid = lax.axis_index("s") * 2 + lax.axis_index("c")
    base = wid * _RPW
    lanes = lax.iota(jnp.int32, 16)
    col_lo = 2 * lanes + ((_MASK_LO >> lanes) & 1)
    col_hi = 2 * lanes + 32 + ((_MASK_HI >> lanes) & 1)

    def chunk_body(k, carry):
        off = base + k * _K
        pltpu.sync_copy(idxs.at[pl.ds(off, _K)], idx_v)
        pltpu.async_copy(table.at[idx_v], rows_v, sem).wait()

        def row_body(r, carry2):
            rv = jnp.full((16,), r, jnp.int32)
            a = plsc.load_gather(rows_v, [rv, col_lo])
            b = plsc.load_gather(rows_v, [rv, col_hi])
            out_v[r, pl.ds(0, 16)] = a
            out_v[r, pl.ds(16, 16)] = b
            return carry2

        lax.fori_loop(0, _K, row_body, 0)
        pltpu.sync_copy(out_v, out.at[pl.ds(off, _K)])
        return carry

    lax.fori_loop(0, _NCH, chunk_body, 0)


def kernel(x):
    table = x.reshape(_BC * _H, _W)
    idxs = (jnp.arange(_BC, dtype=jnp.int32)[:, None] * _H
            + jnp.asarray(_RANDJ, jnp.int32)[None, :]).reshape(-1)
    out = _sc_gather(table, idxs)
    return out.reshape(_B, _C, 32, 32)


# speedup vs baseline: 1.1282x; 1.1282x over previous
import functools

import numpy as np
import jax
import jax.numpy as jnp
from jax import lax
from jax.experimental import pallas as pl
from jax.experimental.pallas import tpu as pltpu
from jax.experimental.pallas import tpu_sc as plsc


_RANDI = np.array([0, 3, 5, 7, 8, 11, 12, 15, 16, 19, 20, 23, 24, 27, 29, 30,
                   33, 35, 37, 39, 41, 43, 44, 47, 49, 51, 53, 54, 56, 59, 60,
                   63], dtype=np.int32)
_RANDJ = np.array([1, 2, 4, 6, 9, 10, 13, 14, 16, 19, 20, 22, 24, 27, 29, 30,
                   33, 34, 37, 39, 41, 42, 44, 46, 48, 50, 52, 54, 56, 59, 60,
                   62], dtype=np.int32)

_B, _C, _H, _W = 1024, 16, 64, 64
_BC = _B * _C
_NROWS = _BC * 32
_NW = 32
_RPW = _NROWS // _NW
_K = 128
_NCH = _RPW // _K

_MASK_LO = int(sum((int(_RANDI[i]) - 2 * i) << i for i in range(16)))
_MASK_HI = int(sum((int(_RANDI[16 + i]) - 2 * (16 + i)) << i for i in range(16)))

_mesh = plsc.VectorSubcoreMesh(core_axis_name="c", subcore_axis_name="s")


_NBUF = 4


@functools.partial(
    pl.kernel,
    out_type=jax.ShapeDtypeStruct((_NROWS, 32), jnp.float32),
    mesh=_mesh,
    compiler_params=pltpu.CompilerParams(needs_layout_passes=False,
                                         use_tc_tiling_on_sc=False),
    scratch_types=[
        pltpu.VMEM((_NCH, _K), jnp.int32),
        pltpu.VMEM((_NBUF, _K, _W), jnp.float32),
        pltpu.VMEM((_NBUF, _K, 32), jnp.float32),
        [pltpu.SemaphoreType.DMA] * _NBUF,
        [pltpu.SemaphoreType.DMA] * _NBUF,
    ],
)
def _sc_gather(table, idx2d, out, idx_all, rows, outs, gsems, osems):
    wid = lax.axis_index("s") * 2 + lax.axis_index("c")
    base = wid * _RPW
    lanes = lax.iota(jnp.int32, 16)
    col_lo = 2 * lanes + ((_MASK_LO >> lanes) & 1)
    col_hi = 2 * lanes + 32 + ((_MASK_HI >> lanes) & 1)

    pltpu.sync_copy(idx2d.at[pl.ds(wid * _NCH, _NCH)], idx_all)

    def fire(k, b):
        pltpu.async_copy(table.at[idx_all.at[k]], rows.at[b], gsems[b])

    for b in range(_NBUF):
        fire(b, b)

    def outer(g, carry):
        for b in range(_NBUF):
            k = g * _NBUF + b
            pltpu.make_async_copy(table.at[idx_all.at[k]], rows.at[b],
                                  gsems[b]).wait()
            @pl.when(k >= _NBUF)
            def _():
                pltpu.make_async_copy(
                    outs.at[b], out.at[pl.ds(base + (k - _NBUF) * _K, _K)],
                    osems[b]).wait()

            rv = rows.at[b]
            ov = outs.at[b]

            def row_body(r, c2):
                rvec = jnp.full((16,), r, jnp.int32)
                ov[r, pl.ds(0, 16)] = plsc.load_gather(rv, [rvec, col_lo])
                ov[r, pl.ds(16, 16)] = plsc.load_gather(rv, [rvec, col_hi])
                return c2

            lax.fori_loop(0, _K, row_body, 0, unroll=4)
            pltpu.async_copy(ov, out.at[pl.ds(base + k * _K, _K)], osems[b])

            @pl.when(k + _NBUF < _NCH)
            def _():
                fire(k + _NBUF, b)
        return carry

    lax.fori_loop(0, _NCH // _NBUF, outer, 0)

    for b in range(_NBUF):
        k = _NCH - _NBUF + b
        pltpu.make_async_copy(outs.at[b], out.at[pl.ds(base + k * _K, _K)],
                              osems[b]).wait()


def kernel(x):
    table = x.reshape(_BC * _H, _W)
    idxs = (jnp.arange(_BC, dtype=jnp.int32)[:, None] * _H
            + jnp.asarray(_RANDJ, jnp.int32)[None, :]).reshape(_NW, _NCH, _K)
    idx2d = idxs.reshape(_NW * _NCH, _K)
    out = _sc_gather(table, idx2d)
    return out.reshape(_B, _C, 32, 32)
